# baseline (speedup 1.0000x reference)
import functools

import jax
import jax.numpy as jnp
from jax import lax
from jax.experimental import pallas as pl
from jax.experimental.pallas import tpu as pltpu
from jax.experimental.pallas import tpu_sc as plsc


def _sc_gather(wte, idx):
    V, D = wte.shape
    B = idx.shape[0]
    info = plsc.get_sparse_core_info()
    NC, NS = info.num_cores, info.num_subcores
    NW = NC * NS
    b_per_w = B // NW

    mesh = plsc.VectorSubcoreMesh(core_axis_name="c", subcore_axis_name="s")

    @functools.partial(
        pl.kernel,
        mesh=mesh,
        out_type=jax.ShapeDtypeStruct((B, D), jnp.float32),
        scratch_types=[
            pltpu.VMEM((b_per_w,), jnp.int32),
            pltpu.VMEM((b_per_w, D), jnp.float32),
            pltpu.SemaphoreType.DMA,
        ],
        compiler_params=pltpu.CompilerParams(use_tc_tiling_on_sc=False),
    )
    def gather_kernel(table_hbm, idx_hbm, out_hbm, idx_v, rows_v, sem):
        wid = lax.axis_index("s") * NC + lax.axis_index("c")
        base = wid * b_per_w
        pltpu.sync_copy(idx_hbm.at[pl.ds(base, b_per_w)], idx_v)
        pltpu.async_copy(table_hbm.at[idx_v], rows_v, sem).wait()
        pltpu.sync_copy(rows_v, out_hbm.at[pl.ds(base, b_per_w)])

    return gather_kernel(wte, idx)


_NTB = 16


def _proj_body(emb_ref, w_ref, out_ref):
    mt = out_ref.shape[1]
    e = emb_ref[...].astype(jnp.bfloat16)
    w = w_ref[...].astype(jnp.bfloat16)
    acc = lax.dot_general(
        e, w,
        dimension_numbers=(((1,), (1,)), ((), ())),
        preferred_element_type=jnp.float32,
    ).astype(jnp.bfloat16)
    out_ref[...] = acc.reshape(mt, 16, _NTB, 128).transpose(2, 0, 1, 3)


def _tc_project(emb, lm_head_w):
    B, D = emb.shape
    V = lm_head_w.shape[0]
    MT = B // 16
    NT = (V + 127) // 128
    slab = _NTB * 128
    grid = (NT + _NTB - 1) // _NTB
    t = pl.pallas_call(
        _proj_body,
        grid=(grid,),
        in_specs=[
            pl.BlockSpec((B, D), lambda i: (0, 0)),
            pl.BlockSpec((slab, D), lambda i: (i, 0)),
        ],
        out_specs=pl.BlockSpec((_NTB, MT, 16, 128), lambda i: (i, 0, 0, 0)),
        out_shape=jax.ShapeDtypeStruct((NT, MT, 16, 128), jnp.bfloat16),
        compiler_params=pltpu.CompilerParams(
            dimension_semantics=("parallel",),
        ),
    )(emb, lm_head_w)
    logits = t.transpose(1, 2, 0, 3).reshape(B, NT * 128)[:, :V]
    return logits.astype(jnp.float32)


def kernel(idx, wte, lm_head_w):
    emb = _sc_gather(wte, idx.astype(jnp.int32))
    return _tc_project(emb, lm_head_w)


# speedup vs baseline: 1.7558x; 1.7558x over previous
import functools

import jax
import jax.numpy as jnp
from jax import lax
from jax.experimental import pallas as pl
from jax.experimental.pallas import tpu as pltpu
from jax.experimental.pallas import tpu_sc as plsc


def _sc_gather_pairs(wte2, idx2):
    D2 = wte2.shape[1]
    B = idx2.shape[0]
    info = plsc.get_sparse_core_info()
    NC, NS = info.num_cores, info.num_subcores
    NW = NC * NS
    b_per_w = B // NW

    mesh = plsc.VectorSubcoreMesh(core_axis_name="c", subcore_axis_name="s")

    @functools.partial(
        pl.kernel,
        mesh=mesh,
        out_type=jax.ShapeDtypeStruct((B, D2), jnp.float32),
        scratch_types=[
            pltpu.VMEM((b_per_w,), jnp.int32),
            pltpu.VMEM((b_per_w, D2), jnp.float32),
            pltpu.SemaphoreType.DMA,
        ],
    )
    def gather_kernel(table_hbm, idx_hbm, out_hbm, idx_v, rows_v, sem):
        wid = lax.axis_index("s") * NC + lax.axis_index("c")
        base = wid * b_per_w
        pltpu.sync_copy(idx_hbm.at[pl.ds(base, b_per_w)], idx_v)
        pltpu.async_copy(table_hbm.at[idx_v], rows_v, sem).wait()
        pltpu.sync_copy(rows_v, out_hbm.at[pl.ds(base, b_per_w)])

    return gather_kernel(wte2, idx2)


_BN = 2048


def _proj_body(emb2_ref, par_ref, w_ref, out_ref):
    D = w_ref.shape[1]
    lo = emb2_ref[:, 0:D]
    hi = emb2_ref[:, D:2 * D]
    e = jnp.where(par_ref[...] != 0, hi, lo).astype(jnp.bfloat16)
    w = w_ref[...].astype(jnp.bfloat16)
    acc = lax.dot_general(
        e, w,
        dimension_numbers=(((1,), (1,)), ((), ())),
        preferred_element_type=jnp.float32,
    )
    out_ref[...] = acc.astype(jnp.bfloat16)


def _tc_project(emb2, parity, lm_head_w):
    B = emb2.shape[0]
    V, D = lm_head_w.shape
    grid = (V + _BN - 1) // _BN
    return pl.pallas_call(
        _proj_body,
        grid=(grid,),
        in_specs=[
            pl.BlockSpec((B, 2 * D), lambda i: (0, 0)),
            pl.BlockSpec((B, 1), lambda i: (0, 0)),
            pl.BlockSpec((_BN, D), lambda i: (i, 0)),
        ],
        out_specs=pl.BlockSpec((B, _BN), lambda i: (0, i)),
        out_shape=jax.ShapeDtypeStruct((B, V), jnp.bfloat16),
        compiler_params=pltpu.CompilerParams(
            dimension_semantics=("parallel",),
        ),
    )(emb2, parity, lm_head_w)


def kernel(idx, wte, lm_head_w):
    V, D = wte.shape
    idx = idx.astype(jnp.int32)
    wte2 = wte.reshape(V // 2, 2 * D)
    emb2 = _sc_gather_pairs(wte2, idx >> 1)
    parity = (idx & 1).reshape(-1, 1)
    return _tc_project(emb2, parity, lm_head_w).astype(jnp.float32)


# speedup vs baseline: 1.7883x; 1.0185x over previous
import functools

import jax
import jax.numpy as jnp
from jax import lax
from jax.experimental import pallas as pl
from jax.experimental.pallas import tpu as pltpu
from jax.experimental.pallas import tpu_sc as plsc


def _sc_gather(wte, idx):
    V, D = wte.shape
    B = idx.shape[0]
    info = plsc.get_sparse_core_info()
    NC, NS = info.num_cores, info.num_subcores
    NW = NC * NS
    b_per_w = B // NW

    mesh = plsc.VectorSubcoreMesh(core_axis_name="c", subcore_axis_name="s")

    @functools.partial(
        pl.kernel,
        mesh=mesh,
        out_type=jax.ShapeDtypeStruct((B, D), jnp.float32),
        scratch_types=[
            pltpu.VMEM((b_per_w,), jnp.int32),
            pltpu.VMEM((b_per_w, D), jnp.float32),
            pltpu.SemaphoreType.DMA,
        ],
        compiler_params=pltpu.CompilerParams(use_tc_tiling_on_sc=False),
    )
    def gather_kernel(table_hbm, idx_hbm, out_hbm, idx_v, rows_v, sem):
        wid = lax.axis_index("s") * NC + lax.axis_index("c")
        base = wid * b_per_w
        pltpu.sync_copy(idx_hbm.at[pl.ds(base, b_per_w)], idx_v)
        pltpu.async_copy(table_hbm.at[idx_v], rows_v, sem).wait()
        pltpu.sync_copy(rows_v, out_hbm.at[pl.ds(base, b_per_w)])

    return gather_kernel(wte, idx)


_BN = 2048


def _proj_body(emb_ref, w_ref, out_ref):
    e = emb_ref[...].astype(jnp.bfloat16)
    w = w_ref[...].astype(jnp.bfloat16)
    acc = lax.dot_general(
        e, w,
        dimension_numbers=(((1,), (1,)), ((), ())),
        preferred_element_type=jnp.float32,
    )
    out_ref[...] = acc.astype(jnp.bfloat16)


def _tc_project(emb, lm_head_w):
    B, D = emb.shape
    V = lm_head_w.shape[0]
    grid = (V + _BN - 1) // _BN
    return pl.pallas_call(
        _proj_body,
        grid=(grid,),
        in_specs=[
            pl.BlockSpec((B, D), lambda i: (0, 0)),
            pl.BlockSpec((_BN, D), lambda i: (i, 0)),
        ],
        out_specs=pl.BlockSpec((B, _BN), lambda i: (0, i)),
        out_shape=jax.ShapeDtypeStruct((B, V), jnp.bfloat16),
        compiler_params=pltpu.CompilerParams(
            dimension_semantics=("parallel",),
        ),
    )(emb, lm_head_w)


def kernel(idx, wte, lm_head_w):
    emb = _sc_gather(wte, idx.astype(jnp.int32))
    return _tc_project(emb, lm_head_w).astype(jnp.float32)
